# Initial kernel scaffold; baseline (speedup 1.0000x reference)
import jax
import jax.numpy as jnp
from jax.experimental import pallas as pl


def kernel(hidden_states, W1, b1, W2, b2, ln_gamma, ln_beta, pattern_memory):
    raise NotImplementedError("write your pallas kernel here")


# speedup vs baseline: 3.9215x; 3.9215x over previous
import jax
import jax.numpy as jnp
from jax.experimental import pallas as pl
from jax.experimental.pallas import tpu as pltpu

D = 768
K = 5
TQ1 = 512
TQ = 256
TM = 2048
PAD = 128


def _mlp_body(x_ref, w1_ref, b1_ref, w2_ref, b2_ref, g_ref, beta_ref, o_ref):
    x = x_ref[...]
    h = jnp.maximum(
        jax.lax.dot_general(x, w1_ref[...], (((1,), (0,)), ((), ())),
                            preferred_element_type=jnp.float32) + b1_ref[...],
        0.0)
    f = jax.lax.dot_general(h, w2_ref[...], (((1,), (0,)), ((), ())),
                            preferred_element_type=jnp.float32) + b2_ref[...]
    mu = jnp.mean(f, axis=-1, keepdims=True)
    var = jnp.mean((f - mu) * (f - mu), axis=-1, keepdims=True)
    o_ref[...] = (f - mu) * jax.lax.rsqrt(var + 1e-5) * g_ref[...] + beta_ref[...]


def _dist_body(q_ref, mem_ref, o_ref, vals_ref, cnts_ref):
    mi = pl.program_id(0)
    qi = pl.program_id(1)
    nm = pl.num_programs(0)
    inf = jnp.float32(jnp.inf)

    q = q_ref[...]
    mem = mem_ref[...]
    g = jax.lax.dot_general(q, mem, (((1,), (1,)), ((), ())),
                            preferred_element_type=jnp.float32)
    mem2 = mem * mem
    ones_r = jnp.ones((8, D), jnp.float32)
    msq = jax.lax.dot_general(ones_r, mem2, (((1,), (1,)), ((), ())),
                              preferred_element_type=jnp.float32)[0:1, :]
    d2 = msq - 2.0 * g

    rows = pl.ds(qi * TQ, TQ)

    @pl.when(mi == 0)
    def _init():
        vals_ref[rows, :] = jnp.full((TQ, PAD), inf, jnp.float32)
        cnts_ref[rows, :] = jnp.zeros((TQ, PAD), jnp.float32)

    rv = vals_ref[rows, :]
    rc = cnts_ref[rows, :]
    aug = jnp.concatenate([d2, rv], axis=1)
    w = jnp.concatenate([jnp.ones_like(d2), rc], axis=1)

    new_vals = []
    new_cnts = []
    for _ in range(K):
        m = jnp.min(aug, axis=1, keepdims=True)
        eq = aug == m
        c = jnp.sum(jnp.where(eq, w, 0.0), axis=1, keepdims=True)
        aug = jnp.where(eq, inf, aug)
        new_vals.append(m)
        new_cnts.append(c)

    lanes = jax.lax.broadcasted_iota(jnp.int32, (TQ, PAD), 1)
    nv = jnp.full((TQ, PAD), inf, jnp.float32)
    nc = jnp.zeros((TQ, PAD), jnp.float32)
    for t in range(K):
        nv = jnp.where(lanes == t, new_vals[t], nv)
        nc = jnp.where(lanes == t, new_cnts[t], nc)
    vals_ref[rows, :] = nv
    cnts_ref[rows, :] = nc

    @pl.when(mi == nm - 1)
    def _finish():
        qsq = jnp.sum(q * q, axis=1, keepdims=True)
        cum = jnp.zeros((TQ, 1), jnp.float32)
        tot = jnp.zeros((TQ, 1), jnp.float32)
        kf = jnp.float32(K)
        for t in range(K):
            take = jnp.clip(kf - cum, 0.0, new_cnts[t])
            tot = tot + jnp.where(take > 0.0, new_vals[t] * take, 0.0)
            cum = cum + new_cnts[t]
        avg = tot / kf + qsq
        res = jax.nn.sigmoid(avg - 1.0)
        o_ref[...] = jnp.broadcast_to(res, (TQ, 8))


def kernel(hidden_states, W1, b1, W2, b2, ln_gamma, ln_beta, pattern_memory):
    B, S, d = hidden_states.shape
    BS = B * S
    M = pattern_memory.shape[0]
    x = hidden_states.reshape(BS, d)

    feats = pl.pallas_call(
        _mlp_body,
        grid=(BS // TQ1,),
        in_specs=[
            pl.BlockSpec((TQ1, d), lambda i: (i, 0)),
            pl.BlockSpec((d, 2 * d), lambda i: (0, 0)),
            pl.BlockSpec((1, 2 * d), lambda i: (0, 0)),
            pl.BlockSpec((2 * d, d), lambda i: (0, 0)),
            pl.BlockSpec((1, d), lambda i: (0, 0)),
            pl.BlockSpec((1, d), lambda i: (0, 0)),
            pl.BlockSpec((1, d), lambda i: (0, 0)),
        ],
        out_specs=pl.BlockSpec((TQ1, d), lambda i: (i, 0)),
        out_shape=jax.ShapeDtypeStruct((BS, d), jnp.float32),
    )(x, W1, b1.reshape(1, -1), W2, b2.reshape(1, -1),
      ln_gamma.reshape(1, -1), ln_beta.reshape(1, -1))

    nm = M // TM
    nq = BS // TQ
    out8 = pl.pallas_call(
        _dist_body,
        grid=(nm, nq),
        in_specs=[
            pl.BlockSpec((TQ, d), lambda mi, qi: (qi, 0)),
            pl.BlockSpec((TM, d), lambda mi, qi: (mi, 0)),
        ],
        out_specs=pl.BlockSpec((TQ, 8), lambda mi, qi: (qi, 0)),
        out_shape=jax.ShapeDtypeStruct((BS, 8), jnp.float32),
        scratch_shapes=[
            pltpu.VMEM((BS, PAD), jnp.float32),
            pltpu.VMEM((BS, PAD), jnp.float32),
        ],
    )(feats, pattern_memory)

    return out8[:, 0].reshape(B, S)


# speedup vs baseline: 4.9952x; 1.2738x over previous
import jax
import jax.numpy as jnp
from jax.experimental import pallas as pl
from jax.experimental.pallas import tpu as pltpu

D = 768
K = 5
TQ1 = 512
TQ = 256
TM = 2048
PAD = 128


def _mlp_body(x_ref, w1_ref, b1_ref, w2_ref, b2_ref, g_ref, beta_ref,
              qn2_ref, qsq_ref):
    x = x_ref[...].astype(jnp.bfloat16)
    h = jnp.maximum(
        jax.lax.dot_general(x, w1_ref[...], (((1,), (0,)), ((), ())),
                            preferred_element_type=jnp.float32) + b1_ref[...],
        0.0)
    f = jax.lax.dot_general(h.astype(jnp.bfloat16), w2_ref[...],
                            (((1,), (0,)), ((), ())),
                            preferred_element_type=jnp.float32) + b2_ref[...]
    mu = jnp.mean(f, axis=-1, keepdims=True)
    var = jnp.mean((f - mu) * (f - mu), axis=-1, keepdims=True)
    q = (f - mu) * jax.lax.rsqrt(var + 1e-5) * g_ref[...] + beta_ref[...]
    qn2_ref[...] = (-2.0 * q).astype(jnp.bfloat16)
    qsq_ref[...] = jnp.broadcast_to(
        jnp.sum(q * q, axis=-1, keepdims=True), qsq_ref.shape)


def _prep_body(mem_ref, memb_ref, msq_ref):
    mem = mem_ref[...]
    mem2 = mem * mem
    ones_r = jnp.ones((8, D), jnp.float32)
    msq_ref[...] = jax.lax.dot_general(ones_r, mem2, (((1,), (1,)), ((), ())),
                                       preferred_element_type=jnp.float32)
    memb_ref[...] = mem.astype(jnp.bfloat16)


def _dist_body(qn2_ref, memb_ref, msq_ref, qsq_ref, o_ref, vals_ref, cnts_ref):
    mi = pl.program_id(0)
    qi = pl.program_id(1)
    nm = pl.num_programs(0)
    inf = jnp.float32(jnp.inf)

    g = jax.lax.dot_general(qn2_ref[...], memb_ref[...],
                            (((1,), (1,)), ((), ())),
                            preferred_element_type=jnp.float32)
    d2 = msq_ref[0:1, :] + g

    rows = pl.ds(qi * TQ, TQ)

    @pl.when(mi == 0)
    def _init():
        vals_ref[rows, :] = jnp.full((TQ, PAD), inf, jnp.float32)
        cnts_ref[rows, :] = jnp.zeros((TQ, PAD), jnp.float32)

    s = [jnp.full((TQ, 128), inf, jnp.float32) for _ in range(K)]
    for j in range(TM // 128):
        c = d2[:, j * 128:(j + 1) * 128]
        for t in range(K):
            lo = jnp.minimum(s[t], c)
            c = jnp.maximum(s[t], c)
            s[t] = lo
    cand = jnp.concatenate(s, axis=1)

    rv = vals_ref[rows, :]
    rc = cnts_ref[rows, :]
    mprev = jnp.full((TQ, 1), -inf, jnp.float32)
    new_vals = []
    new_cnts = []
    for t in range(K):
        cb = jnp.where(cand > mprev, cand, inf)
        sb = jnp.where(rv > mprev, rv, inf)
        m = jnp.minimum(jnp.min(cb, axis=1, keepdims=True),
                        jnp.min(sb, axis=1, keepdims=True))
        cnt = (jnp.sum(jnp.where(cand == m, 1.0, 0.0), axis=1, keepdims=True)
               + jnp.sum(jnp.where(rv == m, rc, 0.0), axis=1, keepdims=True))
        new_vals.append(m)
        new_cnts.append(cnt)
        mprev = m

    lanes = jax.lax.broadcasted_iota(jnp.int32, (TQ, PAD), 1)
    nv = jnp.full((TQ, PAD), inf, jnp.float32)
    nc = jnp.zeros((TQ, PAD), jnp.float32)
    for t in range(K):
        nv = jnp.where(lanes == t, new_vals[t], nv)
        nc = jnp.where(lanes == t, new_cnts[t], nc)
    vals_ref[rows, :] = nv
    cnts_ref[rows, :] = nc

    @pl.when(mi == nm - 1)
    def _finish():
        qsq = qsq_ref[:, 0:1]
        cum = jnp.zeros((TQ, 1), jnp.float32)
        tot = jnp.zeros((TQ, 1), jnp.float32)
        kf = jnp.float32(K)
        for t in range(K):
            take = jnp.clip(kf - cum, 0.0, new_cnts[t])
            tot = tot + jnp.where(take > 0.0, new_vals[t] * take, 0.0)
            cum = cum + new_cnts[t]
        avg = tot / kf + qsq
        res = jax.nn.sigmoid(avg - 1.0)
        o_ref[...] = jnp.broadcast_to(res, (TQ, 8))


def kernel(hidden_states, W1, b1, W2, b2, ln_gamma, ln_beta, pattern_memory):
    B, S, d = hidden_states.shape
    BS = B * S
    M = pattern_memory.shape[0]
    x = hidden_states.reshape(BS, d)

    qn2, qsq = pl.pallas_call(
        _mlp_body,
        grid=(BS // TQ1,),
        in_specs=[
            pl.BlockSpec((TQ1, d), lambda i: (i, 0)),
            pl.BlockSpec((d, 2 * d), lambda i: (0, 0)),
            pl.BlockSpec((1, 2 * d), lambda i: (0, 0)),
            pl.BlockSpec((2 * d, d), lambda i: (0, 0)),
            pl.BlockSpec((1, d), lambda i: (0, 0)),
            pl.BlockSpec((1, d), lambda i: (0, 0)),
            pl.BlockSpec((1, d), lambda i: (0, 0)),
        ],
        out_specs=[
            pl.BlockSpec((TQ1, d), lambda i: (i, 0)),
            pl.BlockSpec((TQ1, 8), lambda i: (i, 0)),
        ],
        out_shape=[
            jax.ShapeDtypeStruct((BS, d), jnp.bfloat16),
            jax.ShapeDtypeStruct((BS, 8), jnp.float32),
        ],
    )(x, W1.astype(jnp.bfloat16), b1.reshape(1, -1),
      W2.astype(jnp.bfloat16), b2.reshape(1, -1),
      ln_gamma.reshape(1, -1), ln_beta.reshape(1, -1))

    nm = M // TM
    memb, msq = pl.pallas_call(
        _prep_body,
        grid=(nm,),
        in_specs=[pl.BlockSpec((TM, d), lambda i: (i, 0))],
        out_specs=[
            pl.BlockSpec((TM, d), lambda i: (i, 0)),
            pl.BlockSpec((8, TM), lambda i: (0, i)),
        ],
        out_shape=[
            jax.ShapeDtypeStruct((M, d), jnp.bfloat16),
            jax.ShapeDtypeStruct((8, M), jnp.float32),
        ],
    )(pattern_memory)

    nq = BS // TQ
    out8 = pl.pallas_call(
        _dist_body,
        grid=(nm, nq),
        in_specs=[
            pl.BlockSpec((TQ, d), lambda mi, qi: (qi, 0)),
            pl.BlockSpec((TM, d), lambda mi, qi: (mi, 0)),
            pl.BlockSpec((8, TM), lambda mi, qi: (0, mi)),
            pl.BlockSpec((TQ, 8), lambda mi, qi: (qi, 0)),
        ],
        out_specs=pl.BlockSpec((TQ, 8), lambda mi, qi: (qi, 0)),
        out_shape=jax.ShapeDtypeStruct((BS, 8), jnp.float32),
        scratch_shapes=[
            pltpu.VMEM((BS, PAD), jnp.float32),
            pltpu.VMEM((BS, PAD), jnp.float32),
        ],
    )(qn2, memb, msq, qsq)

    return out8[:, 0].reshape(B, S)


# speedup vs baseline: 6.3955x; 1.2803x over previous
import jax
import jax.numpy as jnp
from jax.experimental import pallas as pl
from jax.experimental.pallas import tpu as pltpu

D = 768
K = 5
TQ1 = 512
TQ = 256
TM = 2048
LW = 128


def _mlp_body(x_ref, w1_ref, b1_ref, w2_ref, b2_ref, g_ref, beta_ref,
              qn2_ref, qsq_ref):
    x = x_ref[...].astype(jnp.bfloat16)
    h = jnp.maximum(
        jax.lax.dot_general(x, w1_ref[...], (((1,), (0,)), ((), ())),
                            preferred_element_type=jnp.float32) + b1_ref[...],
        0.0)
    f = jax.lax.dot_general(h.astype(jnp.bfloat16), w2_ref[...],
                            (((1,), (0,)), ((), ())),
                            preferred_element_type=jnp.float32) + b2_ref[...]
    mu = jnp.mean(f, axis=-1, keepdims=True)
    var = jnp.mean((f - mu) * (f - mu), axis=-1, keepdims=True)
    q = (f - mu) * jax.lax.rsqrt(var + 1e-5) * g_ref[...] + beta_ref[...]
    qn2_ref[...] = (-2.0 * q).astype(jnp.bfloat16)
    qsq_ref[...] = jnp.broadcast_to(
        jnp.sum(q * q, axis=-1, keepdims=True), qsq_ref.shape)


def _prep_body(mem_ref, memb_ref, msq_ref):
    mem = mem_ref[...]
    mem2 = mem * mem
    ones_r = jnp.ones((8, D), jnp.float32)
    msq_ref[...] = jax.lax.dot_general(ones_r, mem2, (((1,), (1,)), ((), ())),
                                       preferred_element_type=jnp.float32)
    memb_ref[...] = mem.astype(jnp.bfloat16)


def _dist_body(qn2_ref, memb_ref, msq_ref, qsq_ref, o_ref, cand_ref):
    mi = pl.program_id(0)
    qi = pl.program_id(1)
    nm = pl.num_programs(0)
    inf = jnp.float32(jnp.inf)

    g = jax.lax.dot_general(qn2_ref[...], memb_ref[...],
                            (((1,), (1,)), ((), ())),
                            preferred_element_type=jnp.float32)
    d2 = msq_ref[0:1, :] + g

    rows = pl.ds(qi * TQ, TQ)

    @pl.when(mi == 0)
    def _init():
        cand_ref[rows, :] = jnp.full((TQ, K * LW), inf, jnp.float32)

    s = [cand_ref[rows, t * LW:(t + 1) * LW] for t in range(K)]
    for j in range(TM // LW):
        c = d2[:, j * LW:(j + 1) * LW]
        for t in range(K):
            lo = jnp.minimum(s[t], c)
            if t + 1 < K:
                c = jnp.maximum(s[t], c)
            s[t] = lo
    for t in range(K):
        cand_ref[rows, t * LW:(t + 1) * LW] = s[t]

    @pl.when(mi == nm - 1)
    def _finish():
        cand = jnp.concatenate(s, axis=1)
        mprev = jnp.full((TQ, 1), -inf, jnp.float32)
        cum = jnp.zeros((TQ, 1), jnp.float32)
        tot = jnp.zeros((TQ, 1), jnp.float32)
        kf = jnp.float32(K)
        for _ in range(K):
            cb = jnp.where(cand > mprev, cand, inf)
            m = jnp.min(cb, axis=1, keepdims=True)
            cnt = jnp.sum(jnp.where(cand == m, 1.0, 0.0), axis=1,
                          keepdims=True)
            take = jnp.clip(kf - cum, 0.0, cnt)
            tot = tot + jnp.where(take > 0.0, m * take, 0.0)
            cum = cum + cnt
            mprev = m
        avg = tot / kf + qsq_ref[:, 0:1]
        res = jax.nn.sigmoid(avg - 1.0)
        o_ref[...] = jnp.broadcast_to(res, (TQ, 8))


def kernel(hidden_states, W1, b1, W2, b2, ln_gamma, ln_beta, pattern_memory):
    B, S, d = hidden_states.shape
    BS = B * S
    M = pattern_memory.shape[0]
    x = hidden_states.reshape(BS, d)

    qn2, qsq = pl.pallas_call(
        _mlp_body,
        grid=(BS // TQ1,),
        in_specs=[
            pl.BlockSpec((TQ1, d), lambda i: (i, 0)),
            pl.BlockSpec((d, 2 * d), lambda i: (0, 0)),
            pl.BlockSpec((1, 2 * d), lambda i: (0, 0)),
            pl.BlockSpec((2 * d, d), lambda i: (0, 0)),
            pl.BlockSpec((1, d), lambda i: (0, 0)),
            pl.BlockSpec((1, d), lambda i: (0, 0)),
            pl.BlockSpec((1, d), lambda i: (0, 0)),
        ],
        out_specs=[
            pl.BlockSpec((TQ1, d), lambda i: (i, 0)),
            pl.BlockSpec((TQ1, 8), lambda i: (i, 0)),
        ],
        out_shape=[
            jax.ShapeDtypeStruct((BS, d), jnp.bfloat16),
            jax.ShapeDtypeStruct((BS, 8), jnp.float32),
        ],
    )(x, W1.astype(jnp.bfloat16), b1.reshape(1, -1),
      W2.astype(jnp.bfloat16), b2.reshape(1, -1),
      ln_gamma.reshape(1, -1), ln_beta.reshape(1, -1))

    nm = M // TM
    memb, msq = pl.pallas_call(
        _prep_body,
        grid=(nm,),
        in_specs=[pl.BlockSpec((TM, d), lambda i: (i, 0))],
        out_specs=[
            pl.BlockSpec((TM, d), lambda i: (i, 0)),
            pl.BlockSpec((8, TM), lambda i: (0, i)),
        ],
        out_shape=[
            jax.ShapeDtypeStruct((M, d), jnp.bfloat16),
            jax.ShapeDtypeStruct((8, M), jnp.float32),
        ],
    )(pattern_memory)

    nq = BS // TQ
    out8 = pl.pallas_call(
        _dist_body,
        grid=(nm, nq),
        in_specs=[
            pl.BlockSpec((TQ, d), lambda mi, qi: (qi, 0)),
            pl.BlockSpec((TM, d), lambda mi, qi: (mi, 0)),
            pl.BlockSpec((8, TM), lambda mi, qi: (0, mi)),
            pl.BlockSpec((TQ, 8), lambda mi, qi: (qi, 0)),
        ],
        out_specs=pl.BlockSpec((TQ, 8), lambda mi, qi: (qi, 0)),
        out_shape=jax.ShapeDtypeStruct((BS, 8), jnp.float32),
        scratch_shapes=[
            pltpu.VMEM((BS, K * LW), jnp.float32),
        ],
    )(qn2, memb, msq, qsq)

    return out8[:, 0].reshape(B, S)


# speedup vs baseline: 7.9727x; 1.2466x over previous
import jax
import jax.numpy as jnp
from jax.experimental import pallas as pl
from jax.experimental.pallas import tpu as pltpu

D = 768
K = 5
TQ = 256
TM = 2048
LW = 128


def _body(x_ref, w1_ref, b1_ref, w2_ref, b2_ref, g_ref, beta_ref, pm_ref,
          o_ref, qn2_s, qsq_s, memb_s, msq_s, cand_s):
    mi = pl.program_id(0)
    qi = pl.program_id(1)
    nm1 = pl.num_programs(0)
    inf = jnp.float32(jnp.inf)
    rows = pl.ds(qi * TQ, TQ)

    @pl.when(mi == 0)
    def _mlp():
        x = x_ref[...].astype(jnp.bfloat16)
        h = jnp.maximum(
            jax.lax.dot_general(x, w1_ref[...], (((1,), (0,)), ((), ())),
                                preferred_element_type=jnp.float32)
            + b1_ref[...], 0.0)
        f = jax.lax.dot_general(h.astype(jnp.bfloat16), w2_ref[...],
                                (((1,), (0,)), ((), ())),
                                preferred_element_type=jnp.float32) + b2_ref[...]
        mu = jnp.mean(f, axis=-1, keepdims=True)
        var = jnp.mean((f - mu) * (f - mu), axis=-1, keepdims=True)
        q = (f - mu) * jax.lax.rsqrt(var + 1e-5) * g_ref[...] + beta_ref[...]
        qn2_s[rows, :] = (-2.0 * q).astype(jnp.bfloat16)
        qsq_s[rows, :] = jnp.broadcast_to(
            jnp.sum(q * q, axis=-1, keepdims=True), (TQ, 8))
        cand_s[rows, :] = jnp.full((TQ, K * LW), inf, jnp.float32)

    @pl.when((mi >= 1) & (qi == 0))
    def _prep():
        mem = pm_ref[...]
        mem2 = mem * mem
        ones_r = jnp.ones((8, D), jnp.float32)
        msq_s[...] = jax.lax.dot_general(ones_r, mem2,
                                         (((1,), (1,)), ((), ())),
                                         preferred_element_type=jnp.float32)
        memb_s[...] = mem.astype(jnp.bfloat16)

    @pl.when(mi >= 1)
    def _dist():
        g = jax.lax.dot_general(qn2_s[rows, :], memb_s[...],
                                (((1,), (1,)), ((), ())),
                                preferred_element_type=jnp.float32)
        d2 = msq_s[0:1, :] + g

        s = [cand_s[rows, t * LW:(t + 1) * LW] for t in range(K)]
        for j in range(TM // LW):
            c = d2[:, j * LW:(j + 1) * LW]
            for t in range(K):
                lo = jnp.minimum(s[t], c)
                if t + 1 < K:
                    c = jnp.maximum(s[t], c)
                s[t] = lo
        for t in range(K):
            cand_s[rows, t * LW:(t + 1) * LW] = s[t]

        @pl.when(mi == nm1 - 1)
        def _finish():
            cand = jnp.concatenate(s, axis=1)
            mprev = jnp.full((TQ, 1), -inf, jnp.float32)
            cum = jnp.zeros((TQ, 1), jnp.float32)
            tot = jnp.zeros((TQ, 1), jnp.float32)
            kf = jnp.float32(K)
            for _ in range(K):
                cb = jnp.where(cand > mprev, cand, inf)
                m = jnp.min(cb, axis=1, keepdims=True)
                cnt = jnp.sum(jnp.where(cand == m, 1.0, 0.0), axis=1,
                              keepdims=True)
                take = jnp.clip(kf - cum, 0.0, cnt)
                tot = tot + jnp.where(take > 0.0, m * take, 0.0)
                cum = cum + cnt
                mprev = m
            avg = tot / kf + qsq_s[rows, 0:1]
            res = jax.nn.sigmoid(avg - 1.0)
            o_ref[...] = jnp.broadcast_to(res, (TQ, 8))


def kernel(hidden_states, W1, b1, W2, b2, ln_gamma, ln_beta, pattern_memory):
    B, S, d = hidden_states.shape
    BS = B * S
    M = pattern_memory.shape[0]
    x = hidden_states.reshape(BS, d)
    nm = M // TM
    nq = BS // TQ

    out8 = pl.pallas_call(
        _body,
        grid=(nm + 1, nq),
        in_specs=[
            pl.BlockSpec((TQ, d), lambda mi, qi: (jnp.where(mi == 0, qi, 0), 0)),
            pl.BlockSpec((d, 2 * d), lambda mi, qi: (0, 0)),
            pl.BlockSpec((1, 2 * d), lambda mi, qi: (0, 0)),
            pl.BlockSpec((2 * d, d), lambda mi, qi: (0, 0)),
            pl.BlockSpec((1, d), lambda mi, qi: (0, 0)),
            pl.BlockSpec((1, d), lambda mi, qi: (0, 0)),
            pl.BlockSpec((1, d), lambda mi, qi: (0, 0)),
            pl.BlockSpec((TM, d), lambda mi, qi: (jnp.maximum(mi - 1, 0), 0)),
        ],
        out_specs=pl.BlockSpec((TQ, 8), lambda mi, qi: (qi, 0)),
        out_shape=jax.ShapeDtypeStruct((BS, 8), jnp.float32),
        scratch_shapes=[
            pltpu.VMEM((BS, D), jnp.bfloat16),
            pltpu.VMEM((BS, 8), jnp.float32),
            pltpu.VMEM((TM, D), jnp.bfloat16),
            pltpu.VMEM((8, TM), jnp.float32),
            pltpu.VMEM((BS, K * LW), jnp.float32),
        ],
    )(x, W1.astype(jnp.bfloat16), b1.reshape(1, -1),
      W2.astype(jnp.bfloat16), b2.reshape(1, -1),
      ln_gamma.reshape(1, -1), ln_beta.reshape(1, -1), pattern_memory)

    return out8[:, 0].reshape(B, S)


# speedup vs baseline: 8.8108x; 1.1051x over previous
import jax
import jax.numpy as jnp
from jax.experimental import pallas as pl
from jax.experimental.pallas import tpu as pltpu

D = 768
K = 5
TQ = 256
TM = 4096
LW = 128


def _body(x_ref, w1_ref, b1_ref, w2_ref, b2_ref, g_ref, beta_ref, pm_ref,
          o_ref, qn2_s, qsq_s, memb_s, msq_s, cand_s):
    mi = pl.program_id(0)
    qi = pl.program_id(1)
    nm1 = pl.num_programs(0)
    inf = jnp.float32(jnp.inf)
    rows = pl.ds(qi * TQ, TQ)

    @pl.when(mi == 0)
    def _mlp():
        x = x_ref[...].astype(jnp.bfloat16)
        h = jnp.maximum(
            jax.lax.dot_general(x, w1_ref[...], (((1,), (0,)), ((), ())),
                                preferred_element_type=jnp.float32)
            + b1_ref[...], 0.0)
        f = jax.lax.dot_general(h.astype(jnp.bfloat16), w2_ref[...],
                                (((1,), (0,)), ((), ())),
                                preferred_element_type=jnp.float32) + b2_ref[...]
        mu = jnp.mean(f, axis=-1, keepdims=True)
        var = jnp.mean((f - mu) * (f - mu), axis=-1, keepdims=True)
        q = (f - mu) * jax.lax.rsqrt(var + 1e-5) * g_ref[...] + beta_ref[...]
        qn2_s[rows, :] = (-2.0 * q).astype(jnp.bfloat16)
        qsq_s[rows, :] = jnp.broadcast_to(
            jnp.sum(q * q, axis=-1, keepdims=True), (TQ, 8))
        cand_s[rows, :] = jnp.full((TQ, K * LW), inf, jnp.float32)

    @pl.when((mi >= 1) & (qi == 0))
    def _prep():
        mem = pm_ref[...]
        mem2 = mem * mem
        ones_r = jnp.ones((8, D), jnp.float32)
        msq_s[...] = jax.lax.dot_general(ones_r, mem2,
                                         (((1,), (1,)), ((), ())),
                                         preferred_element_type=jnp.float32)
        memb_s[...] = mem.astype(jnp.bfloat16)

    @pl.when(mi >= 1)
    def _dist():
        g = jax.lax.dot_general(qn2_s[rows, :], memb_s[...],
                                (((1,), (1,)), ((), ())),
                                preferred_element_type=jnp.float32)
        s = [cand_s[rows, t * LW:(t + 1) * LW] for t in range(K)]
        for j in range(TM // LW):
            c = msq_s[0:1, j * LW:(j + 1) * LW] + g[:, j * LW:(j + 1) * LW]
            for t in range(K):
                lo = jnp.minimum(s[t], c)
                if t + 1 < K:
                    c = jnp.maximum(s[t], c)
                s[t] = lo
        for t in range(K):
            cand_s[rows, t * LW:(t + 1) * LW] = s[t]

        @pl.when(mi == nm1 - 1)
        def _finish():
            cand = jnp.concatenate(s, axis=1)
            mprev = jnp.full((TQ, 1), -inf, jnp.float32)
            cum = jnp.zeros((TQ, 1), jnp.float32)
            tot = jnp.zeros((TQ, 1), jnp.float32)
            kf = jnp.float32(K)
            for _ in range(K):
                cb = jnp.where(cand > mprev, cand, inf)
                m = jnp.min(cb, axis=1, keepdims=True)
                cnt = jnp.sum(jnp.where(cand == m, 1.0, 0.0), axis=1,
                              keepdims=True)
                take = jnp.clip(kf - cum, 0.0, cnt)
                tot = tot + jnp.where(take > 0.0, m * take, 0.0)
                cum = cum + cnt
                mprev = m
            avg = tot / kf + qsq_s[rows, 0:1]
            res = jax.nn.sigmoid(avg - 1.0)
            o_ref[...] = jnp.broadcast_to(res, (TQ, 8))


def kernel(hidden_states, W1, b1, W2, b2, ln_gamma, ln_beta, pattern_memory):
    B, S, d = hidden_states.shape
    BS = B * S
    M = pattern_memory.shape[0]
    x = hidden_states.reshape(BS, d)
    nm = M // TM
    nq = BS // TQ

    out8 = pl.pallas_call(
        _body,
        grid=(nm + 1, nq),
        in_specs=[
            pl.BlockSpec((TQ, d), lambda mi, qi: (jnp.where(mi == 0, qi, 0), 0)),
            pl.BlockSpec((d, 2 * d), lambda mi, qi: (0, 0)),
            pl.BlockSpec((1, 2 * d), lambda mi, qi: (0, 0)),
            pl.BlockSpec((2 * d, d), lambda mi, qi: (0, 0)),
            pl.BlockSpec((1, d), lambda mi, qi: (0, 0)),
            pl.BlockSpec((1, d), lambda mi, qi: (0, 0)),
            pl.BlockSpec((1, d), lambda mi, qi: (0, 0)),
            pl.BlockSpec((TM, d), lambda mi, qi: (jnp.maximum(mi - 1, 0), 0)),
        ],
        out_specs=pl.BlockSpec((TQ, 8), lambda mi, qi: (qi, 0)),
        out_shape=jax.ShapeDtypeStruct((BS, 8), jnp.float32),
        scratch_shapes=[
            pltpu.VMEM((BS, D), jnp.bfloat16),
            pltpu.VMEM((BS, 8), jnp.float32),
            pltpu.VMEM((TM, D), jnp.bfloat16),
            pltpu.VMEM((8, TM), jnp.float32),
            pltpu.VMEM((BS, K * LW), jnp.float32),
        ],
    )(x, W1.astype(jnp.bfloat16), b1.reshape(1, -1),
      W2.astype(jnp.bfloat16), b2.reshape(1, -1),
      ln_gamma.reshape(1, -1), ln_beta.reshape(1, -1), pattern_memory)

    return out8[:, 0].reshape(B, S)


# speedup vs baseline: 9.4745x; 1.0753x over previous
import jax
import jax.numpy as jnp
from jax.experimental import pallas as pl
from jax.experimental.pallas import tpu as pltpu

D = 768
K = 5
TQ = 256
TM = 4096
LW = 128


def _cmp2(a, b):
    return jnp.minimum(a, b), jnp.maximum(a, b)


def _oem(A, B):
    A = list(A)
    B = list(B)
    if not A:
        return B
    if not B:
        return A
    if len(A) == 1 and len(B) == 1:
        lo, hi = _cmp2(A[0], B[0])
        return [lo, hi]
    O = _oem(A[0::2], B[0::2])
    E = _oem(A[1::2], B[1::2])
    res = [O[0]]
    for i in range(len(E)):
        if i + 1 < len(O):
            lo, hi = _cmp2(E[i], O[i + 1])
            res += [lo, hi]
        else:
            res.append(E[i])
    res += O[len(E) + 1:]
    return res


def _msort(L):
    L = list(L)
    if len(L) <= 1:
        return L
    h = len(L) // 2
    return _oem(_msort(L[:h]), _msort(L[h:]))


def _bottomk(L, k):
    if len(L) <= k:
        return _msort(L)
    h = len(L) // 2
    return _oem(_bottomk(L[:h], k), _bottomk(L[h:], k))[:k]


def _body(x_ref, w1_ref, b1_ref, w2_ref, b2_ref, g_ref, beta_ref, pm_ref,
          o_ref, qn2_s, qsq_s, memb_s, msq_s, cand_s):
    mi = pl.program_id(0)
    qi = pl.program_id(1)
    nm1 = pl.num_programs(0)
    inf = jnp.float32(jnp.inf)
    rows = pl.ds(qi * TQ, TQ)

    @pl.when(mi == 0)
    def _mlp():
        x = x_ref[...].astype(jnp.bfloat16)
        h = jnp.maximum(
            jax.lax.dot_general(x, w1_ref[...], (((1,), (0,)), ((), ())),
                                preferred_element_type=jnp.float32)
            + b1_ref[...], 0.0)
        f = jax.lax.dot_general(h.astype(jnp.bfloat16), w2_ref[...],
                                (((1,), (0,)), ((), ())),
                                preferred_element_type=jnp.float32) + b2_ref[...]
        mu = jnp.mean(f, axis=-1, keepdims=True)
        var = jnp.mean((f - mu) * (f - mu), axis=-1, keepdims=True)
        q = (f - mu) * jax.lax.rsqrt(var + 1e-5) * g_ref[...] + beta_ref[...]
        qn2_s[rows, :] = (-2.0 * q).astype(jnp.bfloat16)
        qsq_s[rows, :] = jnp.broadcast_to(
            jnp.sum(q * q, axis=-1, keepdims=True), (TQ, 8))
        cand_s[rows, :] = jnp.full((TQ, K * LW), inf, jnp.float32)

    @pl.when((mi >= 1) & (qi == 0))
    def _prep():
        mem = pm_ref[...]
        mem2 = mem * mem
        ones_r = jnp.ones((8, D), jnp.float32)
        msq_s[...] = jax.lax.dot_general(ones_r, mem2,
                                         (((1,), (1,)), ((), ())),
                                         preferred_element_type=jnp.float32)
        memb_s[...] = mem.astype(jnp.bfloat16)

    @pl.when(mi >= 1)
    def _dist():
        g = jax.lax.dot_general(qn2_s[rows, :], memb_s[...],
                                (((1,), (1,)), ((), ())),
                                preferred_element_type=jnp.float32)
        chunks = [msq_s[0:1, j * LW:(j + 1) * LW] + g[:, j * LW:(j + 1) * LW]
                  for j in range(TM // LW)]
        c5 = _bottomk(chunks, K)
        s = [cand_s[rows, t * LW:(t + 1) * LW] for t in range(K)]
        s = _oem(s, c5)[:K]
        for t in range(K):
            cand_s[rows, t * LW:(t + 1) * LW] = s[t]

        @pl.when(mi == nm1 - 1)
        def _finish():
            cand = jnp.concatenate(s, axis=1)
            mprev = jnp.full((TQ, 1), -inf, jnp.float32)
            cum = jnp.zeros((TQ, 1), jnp.float32)
            tot = jnp.zeros((TQ, 1), jnp.float32)
            kf = jnp.float32(K)
            for _ in range(K):
                cb = jnp.where(cand > mprev, cand, inf)
                m = jnp.min(cb, axis=1, keepdims=True)
                cnt = jnp.sum(jnp.where(cand == m, 1.0, 0.0), axis=1,
                              keepdims=True)
                take = jnp.clip(kf - cum, 0.0, cnt)
                tot = tot + jnp.where(take > 0.0, m * take, 0.0)
                cum = cum + cnt
                mprev = m
            avg = tot / kf + qsq_s[rows, 0:1]
            res = jax.nn.sigmoid(avg - 1.0)
            o_ref[...] = jnp.broadcast_to(res, (TQ, 8))


def kernel(hidden_states, W1, b1, W2, b2, ln_gamma, ln_beta, pattern_memory):
    B, S, d = hidden_states.shape
    BS = B * S
    M = pattern_memory.shape[0]
    x = hidden_states.reshape(BS, d)
    nm = M // TM
    nq = BS // TQ

    out8 = pl.pallas_call(
        _body,
        grid=(nm + 1, nq),
        in_specs=[
            pl.BlockSpec((TQ, d), lambda mi, qi: (jnp.where(mi == 0, qi, 0), 0)),
            pl.BlockSpec((d, 2 * d), lambda mi, qi: (0, 0)),
            pl.BlockSpec((1, 2 * d), lambda mi, qi: (0, 0)),
            pl.BlockSpec((2 * d, d), lambda mi, qi: (0, 0)),
            pl.BlockSpec((1, d), lambda mi, qi: (0, 0)),
            pl.BlockSpec((1, d), lambda mi, qi: (0, 0)),
            pl.BlockSpec((1, d), lambda mi, qi: (0, 0)),
            pl.BlockSpec((TM, d), lambda mi, qi: (jnp.maximum(mi - 1, 0), 0)),
        ],
        out_specs=pl.BlockSpec((TQ, 8), lambda mi, qi: (qi, 0)),
        out_shape=jax.ShapeDtypeStruct((BS, 8), jnp.float32),
        scratch_shapes=[
            pltpu.VMEM((BS, D), jnp.bfloat16),
            pltpu.VMEM((BS, 8), jnp.float32),
            pltpu.VMEM((TM, D), jnp.bfloat16),
            pltpu.VMEM((8, TM), jnp.float32),
            pltpu.VMEM((BS, K * LW), jnp.float32),
        ],
    )(x, W1.astype(jnp.bfloat16), b1.reshape(1, -1),
      W2.astype(jnp.bfloat16), b2.reshape(1, -1),
      ln_gamma.reshape(1, -1), ln_beta.reshape(1, -1), pattern_memory)

    return out8[:, 0].reshape(B, S)
